# Initial kernel scaffold; baseline (speedup 1.0000x reference)
import jax
import jax.numpy as jnp
from jax.experimental import pallas as pl


def kernel(input_ids, emb_table, W, b):
    raise NotImplementedError("write your pallas kernel here")


# speedup vs baseline: 1.2578x; 1.2578x over previous
import functools

import jax
import jax.numpy as jnp
from jax import lax
from jax.experimental import pallas as pl
from jax.experimental.pallas import tpu as pltpu
from jax.experimental.pallas import tpu_sc as plsc


def _fused_table_body(emb_ref, w_ref, b_ref, out_ref):
    out_ref[...] = (
        jnp.dot(emb_ref[...], w_ref[...], preferred_element_type=jnp.float32)
        + b_ref[...]
    )


def _make_fused_table(vocab, d_out):
    return pl.pallas_call(
        _fused_table_body,
        out_shape=jax.ShapeDtypeStruct((vocab, d_out), jnp.float32),
    )


def _make_gather(n_tokens, d_out, chunk):
    info = plsc.get_sparse_core_info()
    nw = info.num_cores * info.num_subcores
    per_w = n_tokens // nw
    n_chunks = per_w // chunk
    assert per_w % chunk == 0 and n_tokens % nw == 0

    mesh = plsc.VectorSubcoreMesh(core_axis_name="c", subcore_axis_name="s")

    @functools.partial(
        pl.kernel,
        mesh=mesh,
        out_type=jax.ShapeDtypeStruct((n_tokens, d_out), jnp.float32),
        scratch_types=[
            pltpu.VMEM((1, chunk), jnp.int32),
            pltpu.VMEM((chunk, d_out), jnp.float32),
            pltpu.SemaphoreType.DMA,
        ],
    )
    def gather(table_hbm, idx_hbm, out_hbm, idx_v, rows_v, gsem):
        wid = lax.axis_index("s") * info.num_cores + lax.axis_index("c")
        base = wid * per_w

        def body(j, carry):
            start = base + j * chunk
            pltpu.sync_copy(idx_hbm.at[pl.ds(start, chunk)], idx_v.at[0])
            pltpu.async_copy(table_hbm.at[idx_v.at[0]], rows_v, gsem).wait()
            pltpu.sync_copy(rows_v, out_hbm.at[pl.ds(start, chunk)])
            return carry

        lax.fori_loop(0, n_chunks, body, 0)

    return gather


def kernel(input_ids, emb_table, W, b):
    batch, seqlen = input_ids.shape
    vocab, d_in = emb_table.shape
    d_out = W.shape[1]
    n_tokens = batch * seqlen

    fused = _make_fused_table(vocab, d_out)(emb_table, W, b.reshape(1, d_out))
    ids = input_ids.reshape(n_tokens).astype(jnp.int32)
    out = _make_gather(n_tokens, d_out, 128)(fused, ids)
    return out.reshape(batch, seqlen, d_out)
